# baseline (speedup 1.0000x reference)
import functools

import jax
import jax.numpy as jnp
import numpy as np
from jax import lax
from jax.experimental import pallas as pl
from jax.experimental.pallas import tpu as pltpu
from jax.experimental.pallas import tpu_sc as plsc

BATCH = 128
S = 128
NA = 8
O = 512
NF = 2
NCORES = 2
NSUB = 16
NW = NCORES * NSUB
EPW = BATCH // NW
L = 16
RPE = O // S


def _make_noise():
    cpu = jax.devices("cpu")[0]
    with jax.set_mesh(None), jax.default_device(cpu):
        key = jax.random.key(42)
        k0, k1, k2, k3 = jax.random.split(key, 4)
        gB = jnp.stack([
            jax.random.gumbel(k0, (BATCH, S), jnp.float32),
            jax.random.gumbel(k1, (BATCH, S), jnp.float32),
        ])
        gA = jnp.stack([
            jax.random.gumbel(k2, (BATCH, O), jnp.float32),
            jax.random.gumbel(k3, (BATCH, O), jnp.float32),
        ])
        egB = np.asarray(jnp.exp(gB)).reshape(-1, S)
        egA = np.asarray(jnp.exp(gA)).reshape(-1, S)
    return egB, egA


_EGB, _EGA = _make_noise()


def _argmax_flat(chunks, lanes):
    best = jnp.full((L,), -jnp.inf, jnp.float32)
    bidx = jnp.zeros((L,), jnp.int32)
    for c, v in enumerate(chunks):
        take = v > best
        best = jnp.where(take, v, best)
        bidx = jnp.where(take, c * L + lanes, bidx)
    m = jnp.max(best)
    cand = jnp.where(best == m, bidx, jnp.int32(2**30))
    return jnp.min(cand)


def _lane_scalar(vec, idx, lanes):
    return jnp.max(jnp.where(lanes == idx, vec, jnp.int32(-2**31)))


@functools.partial(
    pl.kernel,
    out_type=[
        jax.ShapeDtypeStruct((BATCH,), jnp.float32),
        jax.ShapeDtypeStruct((BATCH,), jnp.float32),
        jax.ShapeDtypeStruct((BATCH,), jnp.float32),
        jax.ShapeDtypeStruct((BATCH,), jnp.float32),
    ],
    mesh=plsc.VectorSubcoreMesh(
        core_axis_name="c", subcore_axis_name="s",
        num_cores=NCORES, num_subcores=NSUB,
    ),
    scratch_types=[
        pltpu.VMEM((NF, L), jnp.float32),
        pltpu.VMEM((NF, L), jnp.int32),
        pltpu.VMEM((NF * EPW, S), jnp.int32),
        pltpu.VMEM((NF * EPW, S), jnp.float32),
        pltpu.VMEM((NF * EPW, S), jnp.float32),
        pltpu.VMEM((NF * EPW * RPE, S), jnp.int32),
        pltpu.VMEM((NF * EPW * RPE, S), jnp.float32),
        pltpu.VMEM((NF * EPW * RPE, S), jnp.float32),
        pltpu.VMEM((2, L), jnp.float32),
        pltpu.VMEM_SHARED((NSUB, 2, L), jnp.float32),
        pltpu.VMEM((2, L), jnp.float32),
        pltpu.VMEM((2, L), jnp.float32),
        pltpu.SemaphoreType.DMA,
        pltpu.SemaphoreType.DMA,
        pltpu.SemaphoreType.DMA,
        pltpu.SemaphoreType.DMA,
    ],
    compiler_params=pltpu.CompilerParams(
        use_tc_tiling_on_sc=False, needs_layout_passes=False,
    ),
)
def _env_step(B0, B1, A0, A1, sT, aT, gB, gA,
              o0, o1, ns0, ns1,
              s_v, a_v, idxB, pB, gB_v, idxA, pA, gA_v,
              resv, shared, nb, outbuf,
              sem_gb, sem_ga, sem_b, sem_a):
    w = lax.axis_index("c") * NSUB + lax.axis_index("s")
    b0 = w * EPW
    lanes = lax.iota(jnp.int32, L)
    Bf = (B0, B1)
    Af = (A0, A1)

    gb_copies, ga_copies = [], []
    for f in range(NF):
        cp = pltpu.make_async_copy(
            gB.at[pl.ds((f * NW + w) * EPW, EPW), :],
            gB_v.at[pl.ds(f * EPW, EPW), :], sem_gb)
        cp.start()
        gb_copies.append(cp)
        cp = pltpu.make_async_copy(
            gA.at[pl.ds((f * NW + w) * (EPW * RPE), EPW * RPE), :],
            gA_v.at[pl.ds(f * (EPW * RPE), EPW * RPE), :], sem_ga)
        cp.start()
        ga_copies.append(cp)

    half = pl.multiple_of((w // 2) * 8, 8)
    loff = (w % 2) * EPW
    for f in range(NF):
        pltpu.sync_copy(sT.at[f, pl.ds(half, 8)], s_v.at[f, pl.ds(0, 8)])
        pltpu.sync_copy(aT.at[f, pl.ds(half, 8)], a_v.at[f, pl.ds(0, 8)])

    for f in range(NF):
        s_row = s_v[f, pl.ds(0, L)].astype(jnp.int32)
        a_row = a_v[f, pl.ds(0, L)]
        for j in range(EPW):
            k = f * EPW + j
            s_k = _lane_scalar(s_row, loff + j, lanes)
            a_k = _lane_scalar(a_row, loff + j, lanes)
            base = (b0 + j) * (S * S * NA) + a_k * S + s_k
            for c in range(S // L):
                idxB[k, pl.ds(c * L, L)] = base + (c * L + lanes) * (S * NA)

    for f in range(NF):
        for j in range(EPW):
            k = f * EPW + j
            pltpu.async_copy(Bf[f].at[idxB.at[k]], pB.at[k], sem_b)

    for cp in gb_copies:
        cp.wait()
    for f in range(NF):
        for j in range(EPW):
            k = f * EPW + j
            pltpu.make_async_copy(Bf[f].at[idxB.at[k]], pB.at[k], sem_b).wait()

    ns_pack = jnp.zeros((L,), jnp.float32)
    for f in range(NF):
        for j in range(EPW):
            k = f * EPW + j
            chunks = [
                pB[k, pl.ds(c * L, L)] * gB_v[k, pl.ds(c * L, L)]
                for c in range(S // L)
            ]
            ns_k = _argmax_flat(chunks, lanes)
            ns_pack = jnp.where(
                lanes == f * 8 + loff + j, ns_k.astype(jnp.float32), ns_pack)
            base2 = (b0 + j) * (O * S) + ns_k
            for q in range(RPE):
                for c in range(S // L):
                    idxA[k * RPE + q, pl.ds(c * L, L)] = (
                        base2 + (q * S + c * L + lanes) * S
                    )
            for q in range(RPE):
                r = k * RPE + q
                pltpu.async_copy(Af[f].at[idxA.at[r]], pA.at[r], sem_a)

    resv[0, ...] = ns_pack

    for cp in ga_copies:
        cp.wait()
    for f in range(NF):
        for r in range(EPW * RPE):
            rr = f * EPW * RPE + r
            pltpu.make_async_copy(Af[f].at[idxA.at[rr]], pA.at[rr], sem_a).wait()

    o_pack = jnp.zeros((L,), jnp.float32)
    for f in range(NF):
        for j in range(EPW):
            k = f * EPW + j
            chunks = [
                pA[k * RPE + q, pl.ds(c * L, L)]
                * gA_v[k * RPE + q, pl.ds(c * L, L)]
                for q in range(RPE)
                for c in range(S // L)
            ]
            o_k = _argmax_flat(chunks, lanes)
            o_pack = jnp.where(
                lanes == f * 8 + loff + j, o_k.astype(jnp.float32), o_pack)

    resv[1, ...] = o_pack

    sid = lax.axis_index("s")
    pltpu.sync_copy(resv, shared.at[sid])
    plsc.subcore_barrier()

    pb = pl.multiple_of((w // 2) * 8, 8)

    @pl.when(w % 2 == 0)
    def _():
        pltpu.sync_copy(shared.at[sid + 1], nb)
        mine_mask = ((lanes >> 2) & 1) == 0
        merged_ns = jnp.where(mine_mask, resv[0, pl.ds(0, L)], nb[0, pl.ds(0, L)])
        merged_o = jnp.where(mine_mask, resv[1, pl.ds(0, L)], nb[1, pl.ds(0, L)])
        outbuf[0, ...] = merged_ns
        outbuf[1, ...] = merged_o
        pltpu.sync_copy(outbuf.at[0, pl.ds(0, 8)], ns0.at[pl.ds(pb, 8)])
        pltpu.sync_copy(outbuf.at[0, pl.ds(8, 8)], ns1.at[pl.ds(pb, 8)])
        pltpu.sync_copy(outbuf.at[1, pl.ds(0, 8)], o0.at[pl.ds(pb, 8)])
        pltpu.sync_copy(outbuf.at[1, pl.ds(8, 8)], o1.at[pl.ds(pb, 8)])


def kernel(B0, B1, A0, A1, state, actions):
    o0, o1, ns0, ns1 = _env_step(
        B0.transpose(0, 1, 3, 2).reshape(-1),
        B1.transpose(0, 1, 3, 2).reshape(-1),
        A0.reshape(-1), A1.reshape(-1),
        state.T, actions.astype(jnp.int32).T,
        _EGB, _EGA,
    )
    return (o0[:, None], o1[:, None], ns0, ns1)


# speedup vs baseline: 1.0548x; 1.0548x over previous
import functools

import jax
import jax.numpy as jnp
import numpy as np
from jax import lax
from jax.experimental import pallas as pl
from jax.experimental.pallas import tpu as pltpu
from jax.experimental.pallas import tpu_sc as plsc

BATCH = 128
S = 128
NA = 8
O = 512
NF = 2
NCORES = 2
NSUB = 16
NW = NCORES * NSUB
EPW = BATCH // NW
L = 16
RPE = O // S


def _make_noise():
    cpu = jax.devices("cpu")[0]
    with jax.set_mesh(None), jax.default_device(cpu):
        key = jax.random.key(42)
        k0, k1, k2, k3 = jax.random.split(key, 4)
        gB = jnp.stack([
            jax.random.gumbel(k0, (BATCH, S), jnp.float32),
            jax.random.gumbel(k1, (BATCH, S), jnp.float32),
        ])
        gA = jnp.stack([
            jax.random.gumbel(k2, (BATCH, O), jnp.float32),
            jax.random.gumbel(k3, (BATCH, O), jnp.float32),
        ])
        egB = np.asarray(jnp.exp(gB)).reshape(-1, S)
        egA = np.asarray(jnp.exp(gA)).reshape(-1, S)
    return egB, egA


_EGB, _EGA = _make_noise()


def _argmax_flat(chunks, lanes):
    best = jnp.full((L,), -jnp.inf, jnp.float32)
    bidx = jnp.zeros((L,), jnp.int32)
    for c, v in enumerate(chunks):
        take = v > best
        best = jnp.where(take, v, best)
        bidx = jnp.where(take, c * L + lanes, bidx)
    m = jnp.max(best)
    cand = jnp.where(best == m, bidx, jnp.int32(2**30))
    return jnp.min(cand)


def _lane_scalar(vec, idx, lanes):
    return jnp.max(jnp.where(lanes == idx, vec, jnp.int32(-2**31)))


@functools.partial(
    pl.kernel,
    out_type=[
        jax.ShapeDtypeStruct((BATCH,), jnp.float32),
        jax.ShapeDtypeStruct((BATCH,), jnp.float32),
        jax.ShapeDtypeStruct((BATCH,), jnp.float32),
        jax.ShapeDtypeStruct((BATCH,), jnp.float32),
    ],
    mesh=plsc.VectorSubcoreMesh(
        core_axis_name="c", subcore_axis_name="s",
        num_cores=NCORES, num_subcores=NSUB,
    ),
    scratch_types=[
        pltpu.VMEM((NF, L), jnp.float32),
        pltpu.VMEM((NF, L), jnp.int32),
        pltpu.VMEM((NF * EPW, S), jnp.int32),
        pltpu.VMEM((NF * EPW, S), jnp.float32),
        pltpu.VMEM((NF * EPW, S), jnp.float32),
        pltpu.VMEM((NF * EPW * RPE, S), jnp.int32),
        pltpu.VMEM((NF * EPW * RPE, S), jnp.float32),
        pltpu.VMEM((NF * EPW * RPE, S), jnp.float32),
        pltpu.VMEM((2, L), jnp.float32),
        pltpu.VMEM_SHARED((NSUB, 2, L), jnp.float32),
        pltpu.VMEM((2, L), jnp.float32),
        pltpu.VMEM((2, L), jnp.float32),
        pltpu.SemaphoreType.DMA,
        pltpu.SemaphoreType.DMA,
        pltpu.SemaphoreType.DMA,
        pltpu.SemaphoreType.DMA,
        pltpu.SemaphoreType.DMA,
    ],
    compiler_params=pltpu.CompilerParams(
        use_tc_tiling_on_sc=False, needs_layout_passes=False,
    ),
)
def _env_step(B0, B1, A0, A1, sT, aT, gB, gA,
              o0, o1, ns0, ns1,
              s_v, a_v, idxB, pB, gB_v, idxA, pA, gA_v,
              resv, shared, nb, outbuf,
              sem_gb, sem_ga, sem_b, sem_a, sem_sa):
    w = lax.axis_index("c") * NSUB + lax.axis_index("s")
    b0 = w * EPW
    lanes = lax.iota(jnp.int32, L)
    Bf = (B0, B1)
    Af = (A0, A1)

    gb_copies, ga_copies = [], []
    for f in range(NF):
        cp = pltpu.make_async_copy(
            gB.at[pl.ds((f * NW + w) * EPW, EPW), :],
            gB_v.at[pl.ds(f * EPW, EPW), :], sem_gb)
        cp.start()
        gb_copies.append(cp)
        cp = pltpu.make_async_copy(
            gA.at[pl.ds((f * NW + w) * (EPW * RPE), EPW * RPE), :],
            gA_v.at[pl.ds(f * (EPW * RPE), EPW * RPE), :], sem_ga)
        cp.start()
        ga_copies.append(cp)

    half = pl.multiple_of((w // 2) * 8, 8)
    loff = (w % 2) * EPW
    sa_copies = [
        pltpu.make_async_copy(
            sT.at[f, pl.ds(half, 8)], s_v.at[f, pl.ds(0, 8)], sem_sa)
        for f in range(NF)
    ] + [
        pltpu.make_async_copy(
            aT.at[f, pl.ds(half, 8)], a_v.at[f, pl.ds(0, 8)], sem_sa)
        for f in range(NF)
    ]
    for cp in sa_copies:
        cp.start()
    for cp in sa_copies:
        cp.wait()

    offB = [(c * L + lanes) * (S * NA) for c in range(S // L)]
    offA = [(c * L + lanes) * S for c in range(S // L)]
    for f in range(NF):
        s_row = s_v[f, pl.ds(0, L)].astype(jnp.int32)
        a_row = a_v[f, pl.ds(0, L)]
        for j in range(EPW):
            k = f * EPW + j
            s_k = _lane_scalar(s_row, loff + j, lanes)
            a_k = _lane_scalar(a_row, loff + j, lanes)
            base = (b0 + j) * (S * S * NA) + a_k * S + s_k
            for c in range(S // L):
                idxB[k, pl.ds(c * L, L)] = base + offB[c]
            pltpu.async_copy(Bf[f].at[idxB.at[k]], pB.at[k], sem_b)

    for cp in gb_copies:
        cp.wait()
    for f in range(NF):
        for j in range(EPW):
            k = f * EPW + j
            pltpu.make_async_copy(Bf[f].at[idxB.at[k]], pB.at[k], sem_b).wait()

    ns_pack = jnp.zeros((L,), jnp.float32)
    for f in range(NF):
        for j in range(EPW):
            k = f * EPW + j
            chunks = [
                pB[k, pl.ds(c * L, L)] * gB_v[k, pl.ds(c * L, L)]
                for c in range(S // L)
            ]
            ns_k = _argmax_flat(chunks, lanes)
            ns_pack = jnp.where(
                lanes == f * 8 + loff + j, ns_k.astype(jnp.float32), ns_pack)
            base2 = (b0 + j) * (O * S) + ns_k
            for q in range(RPE):
                r = k * RPE + q
                base2q = base2 + q * (S * S)
                for c in range(S // L):
                    idxA[r, pl.ds(c * L, L)] = base2q + offA[c]
                pltpu.async_copy(Af[f].at[idxA.at[r]], pA.at[r], sem_a)

    resv[0, ...] = ns_pack

    for cp in ga_copies:
        cp.wait()

    o_pack = jnp.zeros((L,), jnp.float32)
    for f in range(NF):
        for j in range(EPW):
            k = f * EPW + j
            for q in range(RPE):
                r = k * RPE + q
                pltpu.make_async_copy(
                    Af[f].at[idxA.at[r]], pA.at[r], sem_a).wait()
            chunks = [
                pA[k * RPE + q, pl.ds(c * L, L)]
                * gA_v[k * RPE + q, pl.ds(c * L, L)]
                for q in range(RPE)
                for c in range(S // L)
            ]
            o_k = _argmax_flat(chunks, lanes)
            o_pack = jnp.where(
                lanes == f * 8 + loff + j, o_k.astype(jnp.float32), o_pack)

    resv[1, ...] = o_pack

    sid = lax.axis_index("s")
    pltpu.sync_copy(resv, shared.at[sid])
    plsc.subcore_barrier()

    pb = pl.multiple_of((w // 2) * 8, 8)

    @pl.when(w % 2 == 0)
    def _():
        pltpu.sync_copy(shared.at[sid + 1], nb)
        mine_mask = ((lanes >> 2) & 1) == 0
        merged_ns = jnp.where(mine_mask, resv[0, pl.ds(0, L)], nb[0, pl.ds(0, L)])
        merged_o = jnp.where(mine_mask, resv[1, pl.ds(0, L)], nb[1, pl.ds(0, L)])
        outbuf[0, ...] = merged_ns
        outbuf[1, ...] = merged_o
        pltpu.sync_copy(outbuf.at[0, pl.ds(0, 8)], ns0.at[pl.ds(pb, 8)])
        pltpu.sync_copy(outbuf.at[0, pl.ds(8, 8)], ns1.at[pl.ds(pb, 8)])
        pltpu.sync_copy(outbuf.at[1, pl.ds(0, 8)], o0.at[pl.ds(pb, 8)])
        pltpu.sync_copy(outbuf.at[1, pl.ds(8, 8)], o1.at[pl.ds(pb, 8)])


def kernel(B0, B1, A0, A1, state, actions):
    o0, o1, ns0, ns1 = _env_step(
        B0.transpose(0, 1, 3, 2).reshape(-1),
        B1.transpose(0, 1, 3, 2).reshape(-1),
        A0.reshape(-1), A1.reshape(-1),
        state.T, actions.astype(jnp.int32).T,
        _EGB, _EGA,
    )
    return (o0[:, None], o1[:, None], ns0, ns1)


# speedup vs baseline: 1.0652x; 1.0099x over previous
import functools

import jax
import jax.numpy as jnp
import numpy as np
from jax import lax
from jax.experimental import pallas as pl
from jax.experimental.pallas import tpu as pltpu
from jax.experimental.pallas import tpu_sc as plsc

BATCH = 128
S = 128
NA = 8
O = 512
NF = 2
NCORES = 2
NSUB = 16
NW = NCORES * NSUB
EPW = BATCH // NW
L = 16
RPE = O // S


def _make_noise():
    cpu = jax.devices("cpu")[0]
    with jax.set_mesh(None), jax.default_device(cpu):
        key = jax.random.key(42)
        k0, k1, k2, k3 = jax.random.split(key, 4)
        gB = jnp.stack([
            jax.random.gumbel(k0, (BATCH, S), jnp.float32),
            jax.random.gumbel(k1, (BATCH, S), jnp.float32),
        ])
        gA = jnp.stack([
            jax.random.gumbel(k2, (BATCH, O), jnp.float32),
            jax.random.gumbel(k3, (BATCH, O), jnp.float32),
        ])
        egB = np.asarray(jnp.exp(gB)).reshape(-1, S)
        egA = np.asarray(jnp.exp(gA)).reshape(-1, S)
    return egB, egA


_EGB, _EGA = _make_noise()


def _argmax_flat(chunks, lanes):
    best = jnp.full((L,), -jnp.inf, jnp.float32)
    bidx = jnp.zeros((L,), jnp.int32)
    for c, v in enumerate(chunks):
        take = v > best
        best = jnp.where(take, v, best)
        bidx = jnp.where(take, c * L + lanes, bidx)
    m = jnp.max(best)
    cand = jnp.where(best == m, bidx, jnp.int32(2**30))
    return jnp.min(cand)


def _lane_scalar(vec, idx, lanes):
    return jnp.max(jnp.where(lanes == idx, vec, jnp.int32(-2**31)))


@functools.partial(
    pl.kernel,
    out_type=[
        jax.ShapeDtypeStruct((BATCH,), jnp.float32),
        jax.ShapeDtypeStruct((BATCH,), jnp.float32),
        jax.ShapeDtypeStruct((BATCH,), jnp.float32),
        jax.ShapeDtypeStruct((BATCH,), jnp.float32),
    ],
    mesh=plsc.VectorSubcoreMesh(
        core_axis_name="c", subcore_axis_name="s",
        num_cores=NCORES, num_subcores=NSUB,
    ),
    scratch_types=[
        pltpu.VMEM((NF, L), jnp.float32),
        pltpu.VMEM((NF, L), jnp.int32),
        pltpu.VMEM((NF, EPW * S), jnp.int32),
        pltpu.VMEM((NF, EPW * S), jnp.float32),
        pltpu.VMEM((NF * EPW, S), jnp.float32),
        pltpu.VMEM((NF * EPW, O), jnp.int32),
        pltpu.VMEM((NF * EPW, O), jnp.float32),
        pltpu.VMEM((NF * EPW * RPE, S), jnp.float32),
        pltpu.VMEM((2, L), jnp.float32),
        pltpu.VMEM_SHARED((NSUB, 2, L), jnp.float32),
        pltpu.VMEM((2, L), jnp.float32),
        pltpu.VMEM((2, L), jnp.float32),
        pltpu.SemaphoreType.DMA,
        pltpu.SemaphoreType.DMA,
        pltpu.SemaphoreType.DMA,
        pltpu.SemaphoreType.DMA,
        pltpu.SemaphoreType.DMA,
    ],
    compiler_params=pltpu.CompilerParams(
        use_tc_tiling_on_sc=False, needs_layout_passes=False,
    ),
)
def _env_step(B0, B1, A0, A1, sT, aT, gB, gA,
              o0, o1, ns0, ns1,
              s_v, a_v, idxB, pB, gB_v, idxA, pA, gA_v,
              resv, shared, nb, outbuf,
              sem_gb, sem_ga, sem_b, sem_a, sem_sa):
    w = lax.axis_index("c") * NSUB + lax.axis_index("s")
    b0 = w * EPW
    lanes = lax.iota(jnp.int32, L)
    Bf = (B0, B1)
    Af = (A0, A1)

    gb_copies, ga_copies = [], []
    for f in range(NF):
        cp = pltpu.make_async_copy(
            gB.at[pl.ds((f * NW + w) * EPW, EPW), :],
            gB_v.at[pl.ds(f * EPW, EPW), :], sem_gb)
        cp.start()
        gb_copies.append(cp)
        cp = pltpu.make_async_copy(
            gA.at[pl.ds((f * NW + w) * (EPW * RPE), EPW * RPE), :],
            gA_v.at[pl.ds(f * (EPW * RPE), EPW * RPE), :], sem_ga)
        cp.start()
        ga_copies.append(cp)

    half = pl.multiple_of((w // 2) * 8, 8)
    loff = (w % 2) * EPW
    sa_copies = [
        pltpu.make_async_copy(
            sT.at[f, pl.ds(half, 8)], s_v.at[f, pl.ds(0, 8)], sem_sa)
        for f in range(NF)
    ] + [
        pltpu.make_async_copy(
            aT.at[f, pl.ds(half, 8)], a_v.at[f, pl.ds(0, 8)], sem_sa)
        for f in range(NF)
    ]
    for cp in sa_copies:
        cp.start()
    for cp in sa_copies:
        cp.wait()

    offB = [(c * L + lanes) * (S * NA) for c in range(S // L)]
    offA = [(c * L + lanes) * S for c in range(S // L)]
    for f in range(NF):
        s_row = s_v[f, pl.ds(0, L)].astype(jnp.int32)
        a_row = a_v[f, pl.ds(0, L)]
        for j in range(EPW):
            s_k = _lane_scalar(s_row, loff + j, lanes)
            a_k = _lane_scalar(a_row, loff + j, lanes)
            base = (b0 + j) * (S * S * NA) + a_k * S + s_k
            for c in range(S // L):
                idxB[f, pl.ds(j * S + c * L, L)] = base + offB[c]
        pltpu.async_copy(Bf[f].at[idxB.at[f]], pB.at[f], sem_b)

    for cp in gb_copies:
        cp.wait()
    for f in range(NF):
        pltpu.make_async_copy(Bf[f].at[idxB.at[f]], pB.at[f], sem_b).wait()

    ns_pack = jnp.zeros((L,), jnp.float32)
    for f in range(NF):
        for j in range(EPW):
            k = f * EPW + j
            chunks = [
                pB[f, pl.ds(j * S + c * L, L)] * gB_v[k, pl.ds(c * L, L)]
                for c in range(S // L)
            ]
            ns_k = _argmax_flat(chunks, lanes)
            ns_pack = jnp.where(
                lanes == f * 8 + loff + j, ns_k.astype(jnp.float32), ns_pack)
            base2 = (b0 + j) * (O * S) + ns_k
            for q in range(RPE):
                base2q = base2 + q * (S * S)
                for c in range(S // L):
                    idxA[k, pl.ds(q * S + c * L, L)] = base2q + offA[c]
            pltpu.async_copy(Af[f].at[idxA.at[k]], pA.at[k], sem_a)

    resv[0, ...] = ns_pack

    for cp in ga_copies:
        cp.wait()

    o_pack = jnp.zeros((L,), jnp.float32)
    for f in range(NF):
        for j in range(EPW):
            k = f * EPW + j
            pltpu.make_async_copy(Af[f].at[idxA.at[k]], pA.at[k], sem_a).wait()
            chunks = [
                pA[k, pl.ds(q * S + c * L, L)]
                * gA_v[k * RPE + q, pl.ds(c * L, L)]
                for q in range(RPE)
                for c in range(S // L)
            ]
            o_k = _argmax_flat(chunks, lanes)
            o_pack = jnp.where(
                lanes == f * 8 + loff + j, o_k.astype(jnp.float32), o_pack)

    resv[1, ...] = o_pack

    sid = lax.axis_index("s")
    pltpu.sync_copy(resv, shared.at[sid])
    plsc.subcore_barrier()

    pb = pl.multiple_of((w // 2) * 8, 8)

    @pl.when(w % 2 == 0)
    def _():
        pltpu.sync_copy(shared.at[sid + 1], nb)
        mine_mask = ((lanes >> 2) & 1) == 0
        merged_ns = jnp.where(mine_mask, resv[0, pl.ds(0, L)], nb[0, pl.ds(0, L)])
        merged_o = jnp.where(mine_mask, resv[1, pl.ds(0, L)], nb[1, pl.ds(0, L)])
        outbuf[0, ...] = merged_ns
        outbuf[1, ...] = merged_o
        pltpu.sync_copy(outbuf.at[0, pl.ds(0, 8)], ns0.at[pl.ds(pb, 8)])
        pltpu.sync_copy(outbuf.at[0, pl.ds(8, 8)], ns1.at[pl.ds(pb, 8)])
        pltpu.sync_copy(outbuf.at[1, pl.ds(0, 8)], o0.at[pl.ds(pb, 8)])
        pltpu.sync_copy(outbuf.at[1, pl.ds(8, 8)], o1.at[pl.ds(pb, 8)])


def kernel(B0, B1, A0, A1, state, actions):
    o0, o1, ns0, ns1 = _env_step(
        B0.transpose(0, 1, 3, 2).reshape(-1),
        B1.transpose(0, 1, 3, 2).reshape(-1),
        A0.reshape(-1), A1.reshape(-1),
        state.T, actions.astype(jnp.int32).T,
        _EGB, _EGA,
    )
    return (o0[:, None], o1[:, None], ns0, ns1)
